# baseline (speedup 1.0000x reference)
import jax
import jax.numpy as jnp
from jax import lax
from jax.experimental import pallas as pl
from jax.experimental.pallas import tpu as pltpu
from jax.experimental.pallas import tpu_sc as plsc

_NROWS = 16


def _row_check_body(rs_hbm, dep_hbm, rs_out_hbm, flag_hbm, rs_s, flag_s):
    del dep_hbm

    c = lax.axis_index("c")

    @pl.when(c == 0)
    def _():
        pltpu.sync_copy(rs_hbm, rs_s)

        def step(i, acc):
            rt_len = rs_s[i + 1] - rs_s[i]
            rs_len = rs_s[i + 1] - rs_s[i]
            return acc & jnp.where(rt_len == rs_len, 1, 0).astype(jnp.int32)

        flag_s[0] = lax.fori_loop(0, _NROWS, step, jnp.int32(1))
        pltpu.sync_copy(rs_s, rs_out_hbm)
        pltpu.sync_copy(flag_s, flag_hbm)


def _row_lengths_equal_sc(row_splits, dep):
    mesh = plsc.ScalarSubcoreMesh(axis_name="c", num_cores=1)
    rs_out, flags = pl.kernel(
        _row_check_body,
        out_type=(
            jax.ShapeDtypeStruct(row_splits.shape, jnp.int32),
            jax.ShapeDtypeStruct((_NROWS,), jnp.int32),
        ),
        mesh=mesh,
        scratch_types=[
            pltpu.SMEM(row_splits.shape, jnp.int32),
            pltpu.SMEM((_NROWS,), jnp.int32),
        ],
    )(row_splits, dep)
    return rs_out, flags[0].astype(jnp.bool_)


def kernel(values, row_splits):
    vals_out = jnp.copy(values)
    rs_out, flag = _row_lengths_equal_sc(row_splits, vals_out)
    return (vals_out, rs_out, flag)


# speedup vs baseline: 1.0221x; 1.0221x over previous
import jax
import jax.numpy as jnp
from jax import lax
from jax.experimental import pallas as pl
from jax.experimental.pallas import tpu as pltpu
from jax.experimental.pallas import tpu_sc as plsc

_NROWS = 16


def _row_check_body(rs_hbm, dep_hbm, rs_out_hbm, flag_hbm, rs_s, flag_s):
    del dep_hbm

    c = lax.axis_index("c")

    @pl.when(c == 0)
    def _():
        pltpu.sync_copy(rs_hbm, rs_s)

        def step(i, acc):
            rt_len = rs_s[i + 1] - rs_s[i]
            rs_len = rs_s[i + 1] - rs_s[i]
            return acc & jnp.where(rt_len == rs_len, 1, 0).astype(jnp.int32)

        flag_s[0] = lax.fori_loop(0, _NROWS, step, jnp.int32(1))
        pltpu.sync_copy(rs_s, rs_out_hbm)
        pltpu.sync_copy(flag_s, flag_hbm)


def _row_lengths_equal_sc(row_splits, dep):
    mesh = plsc.ScalarSubcoreMesh(axis_name="c", num_cores=1)
    rs_out, flags = pl.kernel(
        _row_check_body,
        out_type=(
            jax.ShapeDtypeStruct(row_splits.shape, jnp.int32),
            jax.ShapeDtypeStruct((_NROWS,), jnp.int32),
        ),
        mesh=mesh,
        scratch_types=[
            pltpu.SMEM(row_splits.shape, jnp.int32),
            pltpu.SMEM((_NROWS,), jnp.int32),
        ],
    )(row_splits, dep)
    return rs_out, flags[0].astype(jnp.bool_)


def kernel(values, row_splits):
    vals_out = jnp.where(values == values, values, jnp.float32(0.0))
    rs_out, flag = _row_lengths_equal_sc(row_splits, vals_out)
    return (vals_out, rs_out, flag)


# speedup vs baseline: 1.0375x; 1.0151x over previous
import jax
import jax.numpy as jnp
from jax import lax
from jax.experimental import pallas as pl
from jax.experimental.pallas import tpu as pltpu
from jax.experimental.pallas import tpu_sc as plsc

_NROWS = 16


def _row_check_body(rs_hbm, dep_hbm, rs_out_hbm, flag_hbm, rs_s, flag_s):
    del dep_hbm

    c = lax.axis_index("c")

    @pl.when(c == 0)
    def _():
        pltpu.sync_copy(rs_hbm, rs_s)

        def step(i, acc):
            rt_len = rs_s[i + 1] - rs_s[i]
            rs_len = rs_s[i + 1] - rs_s[i]
            return acc & jnp.where(rt_len == rs_len, 1, 0).astype(jnp.int32)

        flag_s[0] = lax.fori_loop(0, _NROWS, step, jnp.int32(1))
        pltpu.sync_copy(rs_s, rs_out_hbm)
        pltpu.sync_copy(flag_s, flag_hbm)


def _row_lengths_equal_sc(row_splits, dep):
    mesh = plsc.ScalarSubcoreMesh(axis_name="c", num_cores=1)
    rs_out, flags = pl.kernel(
        _row_check_body,
        out_type=(
            jax.ShapeDtypeStruct(row_splits.shape, jnp.int32),
            jax.ShapeDtypeStruct((_NROWS,), jnp.int32),
        ),
        mesh=mesh,
        scratch_types=[
            pltpu.SMEM(row_splits.shape, jnp.int32),
            pltpu.SMEM((_NROWS,), jnp.int32),
        ],
    )(row_splits, dep)
    return rs_out, flags[0].astype(jnp.bool_)


def _copy_block_body(v_ref, o_ref):
    o_ref[...] = v_ref[...]


def _copy_values_tc(values):
    n, d = values.shape
    block_rows = n
    for cand_rows in (2048, 1024, 512, 256, 128, 64, 32, 16, 8, 4, 2, 1):
        if n % cand_rows == 0:
            block_rows = cand_rows
            break
    return pl.pallas_call(
        _copy_block_body,
        out_shape=jax.ShapeDtypeStruct((n, d), values.dtype),
        grid=(n // block_rows,),
        in_specs=[pl.BlockSpec((block_rows, d), lambda i: (i, 0))],
        out_specs=pl.BlockSpec((block_rows, d), lambda i: (i, 0)),
    )(values)


def kernel(values, row_splits):
    vals_out = _copy_values_tc(values)
    rs_out, flag = _row_lengths_equal_sc(row_splits, row_splits)
    return (vals_out, rs_out, flag)


# speedup vs baseline: 1.0376x; 1.0002x over previous
import jax
import jax.numpy as jnp
from jax import lax
from jax.experimental import pallas as pl
from jax.experimental.pallas import tpu as pltpu
from jax.experimental.pallas import tpu_sc as plsc

_NROWS = 16


def _row_check_body(rs_hbm, rs_out_hbm, flag_hbm, rs_s, flag_s):
    c = lax.axis_index("c")

    @pl.when(c == 0)
    def _():
        pltpu.sync_copy(rs_hbm, rs_s)

        def step(i, acc):
            rt_len = rs_s[i + 1] - rs_s[i]
            rs_len = rs_s[i + 1] - rs_s[i]
            return acc & jnp.where(rt_len == rs_len, 1, 0).astype(jnp.int32)

        flag_s[0] = lax.fori_loop(0, _NROWS, step, jnp.int32(1))
        pltpu.sync_copy(rs_s, rs_out_hbm)
        pltpu.sync_copy(flag_s, flag_hbm)


def _row_lengths_equal_sc(row_splits):
    mesh = plsc.ScalarSubcoreMesh(axis_name="c", num_cores=1)
    rs_out, flags = pl.kernel(
        _row_check_body,
        out_type=(
            jax.ShapeDtypeStruct(row_splits.shape, jnp.int32),
            jax.ShapeDtypeStruct((_NROWS,), jnp.int32),
        ),
        mesh=mesh,
        scratch_types=[
            pltpu.SMEM(row_splits.shape, jnp.int32),
            pltpu.SMEM((_NROWS,), jnp.int32),
        ],
    )(row_splits)
    return rs_out, flags[0].astype(jnp.bool_)


def _copy_block_body(v_ref, o_ref):
    o_ref[...] = v_ref[...]


def _copy_values_tc(values):
    n, d = values.shape
    block_rows = n
    for cand_rows in (2048, 1024, 512, 256, 128, 64, 32, 16, 8, 4, 2, 1):
        if n % cand_rows == 0:
            block_rows = cand_rows
            break
    return pl.pallas_call(
        _copy_block_body,
        out_shape=jax.ShapeDtypeStruct((n, d), values.dtype),
        grid=(n // block_rows,),
        in_specs=[pl.BlockSpec((block_rows, d), lambda i: (i, 0))],
        out_specs=pl.BlockSpec((block_rows, d), lambda i: (i, 0)),
    )(values)


def kernel(values, row_splits):
    vals_out = _copy_values_tc(values)
    rs_out, flag = _row_lengths_equal_sc(row_splits)
    return (vals_out, rs_out, flag)
